# Initial kernel scaffold; baseline (speedup 1.0000x reference)
import jax
import jax.numpy as jnp
from jax.experimental import pallas as pl


def kernel(x_operator, x_table, x_column, x_predicate, x_operation, x_literal, x_numeral, W_op, b_op, W_tab, b_tab, W_col, b_col, W_pred, b_pred, W_oper, b_oper, W_lit, b_lit, W_num, b_num, proj1_W, proj1_b, sem_W1, sem_b1, sem_q1, proj2_W, proj2_b, sem_W2, sem_b2, sem_q2, g1, be1, g2, be2, lin_W, lin_b, e_calledby, e_op_sc_tab, e_tab_sc_op, e_op_fi_pred, e_pred_fi_op, e_op_ob_col, e_col_ob_op, e_op_co_oper, e_oper_fi_op, e_oper_co_pred, e_oper_co_col, e_oper_co_lit, e_lit_co_oper, e_oper_co_num, e_num_co_oper, batch_operator):
    raise NotImplementedError("write your pallas kernel here")


# speedup vs baseline: 2.4266x; 2.4266x over previous
import functools

import jax
import jax.numpy as jnp
from jax import lax
from jax.experimental import pallas as pl
from jax.experimental.pallas import tpu as pltpu
from jax.experimental.pallas import tpu_sc as plsc

F32 = jnp.float32
H = 64
LANES = 16
NSUB = 16
NCORES = 2
CH = 128
TRASH = 256
ZR = 128
EPAD = NSUB * CH

_N = {'operator': 50000, 'table': 2000, 'column': 10000, 'predicate': 20000,
      'operation': 30000, 'literal': 10000, 'numeral': 10000}
_ESPEC = {
    'e_calledby': ('operator', 'operator'),
    'e_op_sc_tab': ('operator', 'table'),
    'e_tab_sc_op': ('table', 'operator'),
    'e_op_fi_pred': ('operator', 'predicate'),
    'e_pred_fi_op': ('predicate', 'operator'),
    'e_op_ob_col': ('operator', 'column'),
    'e_col_ob_op': ('column', 'operator'),
    'e_op_co_oper': ('operator', 'operation'),
    'e_oper_fi_op': ('operation', 'operator'),
    'e_oper_co_pred': ('operation', 'predicate'),
    'e_oper_co_col': ('operation', 'column'),
    'e_oper_co_lit': ('operation', 'literal'),
    'e_lit_co_oper': ('literal', 'operation'),
    'e_oper_co_num': ('operation', 'numeral'),
    'e_num_co_oper': ('numeral', 'operation'),
}

_mesh = plsc.VectorSubcoreMesh(core_axis_name="c", subcore_axis_name="s")
_sc_params = pltpu.CompilerParams(use_tc_tiling_on_sc=False)


def _half(n):
    return -(-((n + 1) // 2) // 128) * 128


def _pad_edges(src, dst, n_src, n_dst):
    e = src.shape[0]
    ep = -(-e // EPAD) * EPAD
    if ep != e:
        k = jnp.arange(ep - e, dtype=jnp.int32)
        src = jnp.concatenate([src, k % n_src])
        dst = jnp.concatenate([dst, n_dst + (k % 256)])
    return src, dst


def _zero_rows(buf, nrows, width):
    def body(r, carry):
        for c in range(width // LANES):
            buf[r, pl.ds(c * LANES, LANES)] = jnp.zeros((LANES,), F32)
        return carry
    lax.fori_loop(0, nrows, body, 0)


def _fill_spmem(acc, acc_rows, width, zbuf, sid):
    per_tile = acc_rows // NSUB
    base = sid * per_tile
    nfull, rem = divmod(per_tile, ZR)
    for k in range(nfull):
        pltpu.sync_copy(zbuf, acc.at[pl.ds(base + k * ZR, ZR)])
    if rem:
        pltpu.sync_copy(zbuf.at[pl.ds(0, rem)],
                        acc.at[pl.ds(base + nfull * ZR, rem)])


def _sc_count_recip(dst_i, n_dst):
    ep = dst_i.shape[0]
    half = _half(n_dst)
    acc_rows = half + TRASH
    t_edges = ep // NSUB
    n_chunks = t_edges // CH

    @functools.partial(
        pl.kernel, mesh=_mesh, compiler_params=_sc_params,
        out_type=jax.ShapeDtypeStruct((2 * half, LANES), F32),
        scratch_types=[
            pltpu.VMEM_SHARED((acc_rows, LANES), F32),
            pltpu.VMEM((CH,), jnp.int32),
            pltpu.VMEM((CH,), jnp.int32),
            pltpu.VMEM((CH, LANES), F32),
            pltpu.VMEM((ZR, LANES), F32),
        ])
    def k(dst_hbm, out_hbm, acc, dstv, ldstv, onesv, zbuf):
        cid = lax.axis_index("c")
        sid = lax.axis_index("s")
        lo = cid * half
        _zero_rows(zbuf, ZR, LANES)
        _fill_spmem(acc, acc_rows, LANES, zbuf, sid)

        def ones_row(r, carry):
            onesv[r, :] = jnp.ones((LANES,), F32)
            return carry
        lax.fori_loop(0, CH, ones_row, 0)
        plsc.subcore_barrier()

        ebase = sid * t_edges

        def chunk(i, carry):
            pltpu.sync_copy(dst_hbm.at[pl.ds(ebase + i * CH, CH)], dstv)

            def lmap(j, c2):
                d = dstv[pl.ds(j * LANES, LANES)]
                inr = (d >= lo) & (d < lo + half)
                ldstv[pl.ds(j * LANES, LANES)] = jnp.where(
                    inr, d - lo, half + (d & (TRASH - 1)))
                return c2
            lax.fori_loop(0, CH // LANES, lmap, 0)
            pltpu.sync_copy(onesv, acc.at[ldstv], add=True)
            return carry
        lax.fori_loop(0, n_chunks, chunk, 0)
        plsc.subcore_barrier()

        rows_pt = half // NSUB
        rbase = sid * rows_pt
        nb, rem = divmod(rows_pt, ZR)
        blocks = [(i * ZR, ZR) for i in range(nb)]
        if rem:
            blocks.append((nb * ZR, rem))
        for start, nrows in blocks:
            pltpu.sync_copy(acc.at[pl.ds(rbase + start, nrows)],
                            zbuf.at[pl.ds(0, nrows)])

            def srow(r, carry):
                c = zbuf[r, :]
                zbuf[r, :] = 1.0 / jnp.maximum(c, 1.0)
                return carry
            lax.fori_loop(0, nrows, srow, 0)
            pltpu.sync_copy(zbuf.at[pl.ds(0, nrows)],
                            out_hbm.at[pl.ds(lo + rbase + start, nrows)])

    return k(dst_i)


def _sc_mean_agg(z_src, src_i, dst_i, recip, n_dst):
    ep = src_i.shape[0]
    half = _half(n_dst)
    acc_rows = half + TRASH
    t_edges = ep // NSUB
    n_chunks = t_edges // CH

    @functools.partial(
        pl.kernel, mesh=_mesh, compiler_params=_sc_params,
        out_type=jax.ShapeDtypeStruct((2 * half, H), F32),
        scratch_types=[
            pltpu.VMEM_SHARED((acc_rows, H), F32),
            pltpu.VMEM((CH,), jnp.int32),
            pltpu.VMEM((CH,), jnp.int32),
            pltpu.VMEM((CH,), jnp.int32),
            pltpu.VMEM((CH, H), F32),
            pltpu.VMEM((ZR, H), F32),
            pltpu.VMEM((ZR, LANES), F32),
            pltpu.SemaphoreType.DMA,
        ])
    def k(z_hbm, src_hbm, dst_hbm, recip_hbm, out_hbm,
          acc, srcv, dstv, ldstv, rowsv, zbuf, rcpv, sem):
        cid = lax.axis_index("c")
        sid = lax.axis_index("s")
        lo = cid * half
        _zero_rows(zbuf, ZR, H)
        _fill_spmem(acc, acc_rows, H, zbuf, sid)
        plsc.subcore_barrier()

        ebase = sid * t_edges

        def chunk(i, carry):
            cb = ebase + i * CH
            pltpu.sync_copy(src_hbm.at[pl.ds(cb, CH)], srcv)
            pltpu.sync_copy(dst_hbm.at[pl.ds(cb, CH)], dstv)

            def lmap(j, c2):
                d = dstv[pl.ds(j * LANES, LANES)]
                inr = (d >= lo) & (d < lo + half)
                ldstv[pl.ds(j * LANES, LANES)] = jnp.where(
                    inr, d - lo, half + (d & (TRASH - 1)))
                return c2
            lax.fori_loop(0, CH // LANES, lmap, 0)
            pltpu.async_copy(z_hbm.at[srcv], rowsv, sem).wait()
            pltpu.sync_copy(rowsv, acc.at[ldstv], add=True)
            return carry
        lax.fori_loop(0, n_chunks, chunk, 0)
        plsc.subcore_barrier()

        rows_pt = half // NSUB
        rbase = sid * rows_pt
        nb, rem = divmod(rows_pt, ZR)
        blocks = [(i * ZR, ZR) for i in range(nb)]
        if rem:
            blocks.append((nb * ZR, rem))
        for start, nrows in blocks:
            pltpu.sync_copy(acc.at[pl.ds(rbase + start, nrows)],
                            zbuf.at[pl.ds(0, nrows)])
            pltpu.sync_copy(recip_hbm.at[pl.ds(lo + rbase + start, nrows)],
                            rcpv.at[pl.ds(0, nrows)])

            def srow(r, carry):
                rv = rcpv[r, :]
                for c in range(H // LANES):
                    zbuf[r, pl.ds(c * LANES, LANES)] = (
                        zbuf[r, pl.ds(c * LANES, LANES)] * rv)
                return carry
            lax.fori_loop(0, nrows, srow, 0)
            pltpu.sync_copy(zbuf.at[pl.ds(0, nrows)],
                            out_hbm.at[pl.ds(lo + rbase + start, nrows)])

    return k(z_src, src_i, dst_i, recip)


def _tc_embed_proj(x, w, b, pw, pb):
    n, fd = x.shape
    blk = 2000

    def body(x_ref, w_ref, b_ref, pw_ref, pb_ref, o_ref):
        h = jnp.dot(x_ref[...], w_ref[...], preferred_element_type=F32)
        h = h + b_ref[...]
        o_ref[...] = jnp.dot(h, pw_ref[...],
                             preferred_element_type=F32) + pb_ref[...]

    return pl.pallas_call(
        body,
        grid=(n // blk,),
        in_specs=[
            pl.BlockSpec((blk, fd), lambda i: (i, 0)),
            pl.BlockSpec((fd, H), lambda i: (0, 0)),
            pl.BlockSpec((1, H), lambda i: (0, 0)),
            pl.BlockSpec((H, H), lambda i: (0, 0)),
            pl.BlockSpec((1, H), lambda i: (0, 0)),
        ],
        out_specs=pl.BlockSpec((blk, H), lambda i: (i, 0)),
        out_shape=jax.ShapeDtypeStruct((n, H), F32),
    )(x, w, b.reshape(1, H), pw, pb.reshape(1, H))


def _tc_sem_partial(zs, sw, sb, n_valid):
    p_cnt = len(zs)
    npad = zs[0].shape[0]
    blk = 512
    grid = npad // blk

    def body(*refs):
        z_refs = refs[:p_cnt]
        sw_ref, sb_ref, o_ref = refs[p_cnt], refs[p_cnt + 1], refs[p_cnt + 2]
        i = pl.program_id(0)
        rows = i * blk + lax.broadcasted_iota(jnp.int32, (blk, H), 0)
        msk = rows < n_valid
        parts = []
        for p in range(p_cnt):
            t = jnp.tanh(jnp.dot(z_refs[p][...], sw_ref[...],
                                 preferred_element_type=F32) + sb_ref[...])
            t = jnp.where(msk, t, 0.0)
            parts.append(jnp.sum(t, axis=0, keepdims=True))
        contrib = jnp.concatenate(parts, axis=0)

        @pl.when(i == 0)
        def _():
            o_ref[...] = contrib

        @pl.when(i != 0)
        def _():
            o_ref[...] = o_ref[...] + contrib

    return pl.pallas_call(
        body,
        grid=(grid,),
        in_specs=[pl.BlockSpec((blk, H), lambda i: (i, 0))] * p_cnt + [
            pl.BlockSpec((H, H), lambda i: (0, 0)),
            pl.BlockSpec((1, H), lambda i: (0, 0)),
        ],
        out_specs=pl.BlockSpec((p_cnt, H), lambda i: (0, 0)),
        out_shape=jax.ShapeDtypeStruct((p_cnt, H), F32),
    )(*zs, sw, sb.reshape(1, H))


def _tc_combine(zs, sums, sq, g, be, n_valid, pw=None, pb=None):
    p_cnt = len(zs)
    npad = zs[0].shape[0]
    blk = 512
    grid = npad // blk
    fuse = pw is not None

    def body(*refs):
        z_refs = refs[:p_cnt]
        rest = refs[p_cnt:]
        sums_ref, sq_ref, g_ref, be_ref = rest[0], rest[1], rest[2], rest[3]
        if fuse:
            pw_ref, pb_ref, o_ref = rest[4], rest[5], rest[6]
        else:
            o_ref = rest[4]
        s = sums_ref[...] * (1.0 / n_valid)
        sc = jnp.sum(s * sq_ref[...], axis=1, keepdims=True)
        m = jnp.max(sc)
        e = jnp.exp(sc - m)
        beta = e / jnp.sum(e)
        y = beta[0, 0] * z_refs[0][...]
        for p in range(1, p_cnt):
            y = y + beta[p, 0] * z_refs[p][...]
        y = jnp.where(y > 0, y, jnp.exp(jnp.minimum(y, 0.0)) - 1.0)
        mu = jnp.mean(y, axis=1, keepdims=True)
        var = jnp.mean((y - mu) ** 2, axis=1, keepdims=True)
        y = (y - mu) / jnp.sqrt(var + 1e-5) * g_ref[...] + be_ref[...]
        if fuse:
            y = jnp.dot(y, pw_ref[...],
                        preferred_element_type=F32) + pb_ref[...]
        o_ref[...] = y

    in_specs = [pl.BlockSpec((blk, H), lambda i: (i, 0))] * p_cnt + [
        pl.BlockSpec((p_cnt, H), lambda i: (0, 0)),
        pl.BlockSpec((1, H), lambda i: (0, 0)),
        pl.BlockSpec((1, H), lambda i: (0, 0)),
        pl.BlockSpec((1, H), lambda i: (0, 0)),
    ]
    args = list(zs) + [sums, sq.reshape(1, H), g.reshape(1, H),
                       be.reshape(1, H)]
    if fuse:
        in_specs += [pl.BlockSpec((H, H), lambda i: (0, 0)),
                     pl.BlockSpec((1, H), lambda i: (0, 0))]
        args += [pw, pb.reshape(1, H)]
    return pl.pallas_call(
        body,
        grid=(grid,),
        in_specs=in_specs,
        out_specs=pl.BlockSpec((blk, H), lambda i: (i, 0)),
        out_shape=jax.ShapeDtypeStruct((npad, H), F32),
    )(*args)


def _tc_final(pooled, lin_w, lin_b):
    def body(p_ref, w_ref, b_ref, o_ref):
        o_ref[...] = jnp.dot(p_ref[...], w_ref[...],
                             preferred_element_type=F32) + b_ref[...]

    out = pl.pallas_call(
        body,
        out_shape=jax.ShapeDtypeStruct((pooled.shape[0], 1), F32),
    )(pooled, lin_w, lin_b.reshape(1, 1))
    return out[:, 0]


def _layer(hp, edges, recips):
    def agg(z, en):
        src, dst = edges[en]
        return _sc_mean_agg(z, src, dst, recips[en], _N[_ESPEC[en][1]])

    z_cb = agg(hp, 'e_calledby')
    z_cb2 = agg(z_cb, 'e_calledby')
    z_tab = agg(agg(hp, 'e_op_sc_tab'), 'e_tab_sc_op')
    z_pred = agg(agg(hp, 'e_op_fi_pred'), 'e_pred_fi_op')
    z_col = agg(agg(hp, 'e_op_ob_col'), 'e_col_ob_op')
    a = agg(hp, 'e_op_co_oper')
    z_op1 = agg(a, 'e_oper_fi_op')
    z_op2 = agg(agg(a, 'e_oper_co_pred'), 'e_pred_fi_op')
    z_op3 = agg(agg(a, 'e_oper_co_col'), 'e_col_ob_op')
    z_op4 = agg(agg(agg(a, 'e_oper_co_lit'), 'e_lit_co_oper'), 'e_oper_fi_op')
    z_op5 = agg(agg(agg(a, 'e_oper_co_num'), 'e_num_co_oper'), 'e_oper_fi_op')
    return [z_cb, z_tab, z_pred, z_col, z_op1, z_op2, z_op3, z_op4, z_op5,
            z_cb2]


def kernel(x_operator, x_table, x_column, x_predicate, x_operation, x_literal,
           x_numeral, W_op, b_op, W_tab, b_tab, W_col, b_col, W_pred, b_pred,
           W_oper, b_oper, W_lit, b_lit, W_num, b_num, proj1_W, proj1_b,
           sem_W1, sem_b1, sem_q1, proj2_W, proj2_b, sem_W2, sem_b2, sem_q2,
           g1, be1, g2, be2, lin_W, lin_b, e_calledby, e_op_sc_tab,
           e_tab_sc_op, e_op_fi_pred, e_pred_fi_op, e_op_ob_col, e_col_ob_op,
           e_op_co_oper, e_oper_fi_op, e_oper_co_pred, e_oper_co_col,
           e_oper_co_lit, e_lit_co_oper, e_oper_co_num, e_num_co_oper,
           batch_operator):
    raw = {'e_calledby': e_calledby, 'e_op_sc_tab': e_op_sc_tab,
           'e_tab_sc_op': e_tab_sc_op, 'e_op_fi_pred': e_op_fi_pred,
           'e_pred_fi_op': e_pred_fi_op, 'e_op_ob_col': e_op_ob_col,
           'e_col_ob_op': e_col_ob_op, 'e_op_co_oper': e_op_co_oper,
           'e_oper_fi_op': e_oper_fi_op, 'e_oper_co_pred': e_oper_co_pred,
           'e_oper_co_col': e_oper_co_col, 'e_oper_co_lit': e_oper_co_lit,
           'e_lit_co_oper': e_lit_co_oper, 'e_oper_co_num': e_oper_co_num,
           'e_num_co_oper': e_num_co_oper}
    edges, recips = {}, {}
    for en, (st, dt) in _ESPEC.items():
        src, dst = _pad_edges(raw[en][0], raw[en][1], _N[st], _N[dt])
        edges[en] = (src, dst)
        recips[en] = _sc_count_recip(dst, _N[dt])

    n_op = _N['operator']

    hp1 = _tc_embed_proj(x_operator, W_op, b_op, proj1_W, proj1_b)
    zs1 = _layer(hp1, edges, recips)
    sums1 = _tc_sem_partial(zs1, sem_W1, sem_b1, n_op)
    hp2 = _tc_combine(zs1, sums1, sem_q1, g1, be1, n_op,
                      pw=proj2_W, pb=proj2_b)

    zs2 = _layer(hp2, edges, recips)
    sums2 = _tc_sem_partial(zs2, sem_W2, sem_b2, n_op)
    h2 = _tc_combine(zs2, sums2, sem_q2, g2, be2, n_op)

    nb = 512
    pool_src, pool_dst = _pad_edges(
        jnp.arange(n_op, dtype=jnp.int32), batch_operator.astype(jnp.int32),
        n_op, nb)
    pool_recip = _sc_count_recip(pool_dst, nb)
    pooled = _sc_mean_agg(h2, pool_src, pool_dst, pool_recip, nb)[:nb]
    return _tc_final(pooled, lin_W, lin_b)
